# baseline (speedup 1.0000x reference)
import functools

import jax
import jax.numpy as jnp
from jax import lax
from jax.experimental import pallas as pl
from jax.experimental.pallas import tpu as pltpu
from jax.experimental.pallas import tpu_sc as plsc

_B = 8
_N = 2048
_R = 1024
_NB = _N // _R
_NS = 16
_PW = _N // _NS
_PM = 128


def _sc_body(idx_hbm, v_hbm, dval_hbm, hit_hbm,
             idxw, jlw, jlsh, mrg, vloc, dv_loc, ht_loc):
    c = lax.axis_index("c")
    s = lax.axis_index("s")
    jbase = s * _PW

    pltpu.sync_copy(idx_hbm.at[pl.ds(jbase, _PW)], idxw)
    neg1 = jnp.full((16,), -1, jnp.int32)
    lane = lax.iota(jnp.int32, 16)
    for i in range(_N // 16):
        jlw[pl.ds(i * 16, 16)] = neg1
    for ck in range(_PW // 16):
        pv = idxw[pl.ds(ck * 16, 16)]
        jv = jbase + ck * 16 + lane
        for l in range(16):
            plsc.store_scatter(jlw, [pv], jv, mask=lane == l)

    pltpu.sync_copy(jlw, jlsh.at[pl.ds(s * _N, _N)])
    plsc.subcore_barrier()

    @pl.when(s < _R // _PM)
    def _merge():
        mbase = c * _R + s * _PM
        for t in range(_NS):
            pltpu.sync_copy(jlsh.at[pl.ds(t * _N + mbase, _PM)],
                            mrg.at[pl.ds(t * _PM, _PM)])
        pltpu.sync_copy(v_hbm, vloc)

        for ck in range(_PM // 16):
            acc = neg1
            for t in range(_NS):
                acc = jnp.maximum(acc, mrg[pl.ds(t * _PM + ck * 16, 16)])
            hitv = acc >= 0
            jc = jnp.maximum(acc, 0)
            hti = jnp.where(hitv, jnp.full((16,), 1, jnp.int32),
                            jnp.full((16,), 0, jnp.int32))
            for b in range(_B):
                bvec = jnp.full((16,), b, jnp.int32)
                val = plsc.load_gather(vloc, [bvec, jc])
                dv_loc[b, pl.ds(ck * 16, 16)] = val
                ht_loc[b, pl.ds(ck * 16, 16)] = hti
        pltpu.sync_copy(dv_loc, dval_hbm.at[c, :, pl.ds(s * _PM, _PM)])
        pltpu.sync_copy(ht_loc, hit_hbm.at[c, :, pl.ds(s * _PM, _PM)])


_sc_resolve = pl.kernel(
    _sc_body,
    out_type=(
        jax.ShapeDtypeStruct((_NB, _B, _R), jnp.float32),
        jax.ShapeDtypeStruct((_NB, _B, _R), jnp.int32),
    ),
    mesh=plsc.VectorSubcoreMesh(core_axis_name="c", subcore_axis_name="s"),
    compiler_params=pltpu.CompilerParams(needs_layout_passes=False),
    scratch_types=[
        pltpu.VMEM((_PW,), jnp.int32),
        pltpu.VMEM((_N,), jnp.int32),
        pltpu.VMEM_SHARED((_NS * _N,), jnp.int32),
        pltpu.VMEM((_NS * _PM,), jnp.int32),
        pltpu.VMEM((_B, _N), jnp.float32),
        pltpu.VMEM((_B, _PM), jnp.float32),
        pltpu.VMEM((_B, _PM), jnp.int32),
    ],
)


def _tc_body(dval_ref, hit_ref, t_ref, o_ref):
    b = pl.program_id(0)
    r = pl.program_id(1)
    r0 = r * _R

    dv = dval_ref[0, b, :]
    hv = hit_ref[0, 0, :]

    p_col = r0 + lax.broadcasted_iota(jnp.int32, (_R, _N), 0)
    col = lax.broadcasted_iota(jnp.int32, (_R, _N), 1)
    diag = (col == p_col) & (hv[:, None] > 0)
    o_ref[0, :, :] = jnp.where(diag, dv[:, None], t_ref[0, :, :])


@jax.jit
def kernel(t, idx, v):
    idx32 = idx.astype(jnp.int32)
    dval, hit = _sc_resolve(idx32, v)
    out = pl.pallas_call(
        _tc_body,
        grid=(_B, _NB),
        in_specs=[
            pl.BlockSpec((1, _B, _R), lambda b, r: (r, 0, 0)),
            pl.BlockSpec((1, _B, _R), lambda b, r: (r, 0, 0)),
            pl.BlockSpec((1, _R, _N), lambda b, r: (b, r, 0)),
        ],
        out_specs=pl.BlockSpec((1, _R, _N), lambda b, r: (b, r, 0)),
        out_shape=jax.ShapeDtypeStruct((_B, _N, _N), jnp.float32),
    )(dval, hit, t)
    return out


# speedup vs baseline: 1.0193x; 1.0193x over previous
import functools

import jax
import jax.numpy as jnp
from jax import lax
from jax.experimental import pallas as pl
from jax.experimental.pallas import tpu as pltpu
from jax.experimental.pallas import tpu_sc as plsc

_B = 8
_N = 2048
_R = 1024
_NB = _N // _R
_NS = 16
_PW = _N // _NS
_PM = 128


def _sc_body(idx_hbm, v_hbm, dval_hbm, hit_hbm,
             idxw, jlw, jlsh, mrg, vloc, dv_loc, ht_loc):
    c = lax.axis_index("c")
    s = lax.axis_index("s")
    jbase = s * _PW

    pltpu.sync_copy(idx_hbm.at[pl.ds(jbase, _PW)], idxw)
    neg1 = jnp.full((16,), -1, jnp.int32)
    lane = lax.iota(jnp.int32, 16)
    for i in range(_N // 16):
        jlw[pl.ds(i * 16, 16)] = neg1
    for ck in range(_PW // 16):
        pv = idxw[pl.ds(ck * 16, 16)]
        jv = jbase + ck * 16 + lane
        for l in range(16):
            plsc.store_scatter(jlw, [pv], jv, mask=lane == l)

    pltpu.sync_copy(jlw, jlsh.at[pl.ds(s * _N, _N)])
    plsc.subcore_barrier()

    @pl.when(s < _R // _PM)
    def _merge():
        mbase = c * _R + s * _PM
        for t in range(_NS):
            pltpu.sync_copy(jlsh.at[pl.ds(t * _N + mbase, _PM)],
                            mrg.at[pl.ds(t * _PM, _PM)])
        pltpu.sync_copy(v_hbm, vloc)

        for ck in range(_PM // 16):
            acc = neg1
            for t in range(_NS):
                acc = jnp.maximum(acc, mrg[pl.ds(t * _PM + ck * 16, 16)])
            hitv = acc >= 0
            jc = jnp.maximum(acc, 0)
            hti = jnp.where(hitv, jnp.full((16,), 1, jnp.int32),
                            jnp.full((16,), 0, jnp.int32))
            for b in range(_B):
                bvec = jnp.full((16,), b, jnp.int32)
                val = plsc.load_gather(vloc, [bvec, jc])
                dv_loc[b, pl.ds(ck * 16, 16)] = val
                ht_loc[b, pl.ds(ck * 16, 16)] = hti
        pltpu.sync_copy(dv_loc, dval_hbm.at[c, :, pl.ds(s * _PM, _PM)])
        pltpu.sync_copy(ht_loc, hit_hbm.at[c, :, pl.ds(s * _PM, _PM)])


_sc_resolve = pl.kernel(
    _sc_body,
    out_type=(
        jax.ShapeDtypeStruct((_NB, _B, _R), jnp.float32),
        jax.ShapeDtypeStruct((_NB, _B, _R), jnp.int32),
    ),
    mesh=plsc.VectorSubcoreMesh(core_axis_name="c", subcore_axis_name="s"),
    compiler_params=pltpu.CompilerParams(needs_layout_passes=False),
    scratch_types=[
        pltpu.VMEM((_PW,), jnp.int32),
        pltpu.VMEM((_N,), jnp.int32),
        pltpu.VMEM_SHARED((_NS * _N,), jnp.int32),
        pltpu.VMEM((_NS * _PM,), jnp.int32),
        pltpu.VMEM((_B, _N), jnp.float32),
        pltpu.VMEM((_B, _PM), jnp.float32),
        pltpu.VMEM((_B, _PM), jnp.int32),
    ],
)


def _tc_body(dval_ref, hit_ref, t_ref, o_ref):
    b = pl.program_id(0)
    r = pl.program_id(1)
    r0 = r * _R

    dv = dval_ref[0, b, :]
    hv = hit_ref[0, 0, :]

    o_ref[0, :, :] = t_ref[0, :, :]
    row = lax.broadcasted_iota(jnp.int32, (_R, _R), 0)
    col = lax.broadcasted_iota(jnp.int32, (_R, _R), 1)
    diag = (col == row) & (hv[:, None] > 0)
    o_ref[0, :, pl.ds(r0, _R)] = jnp.where(diag, dv[:, None],
                                           t_ref[0, :, pl.ds(r0, _R)])


@jax.jit
def kernel(t, idx, v):
    idx32 = idx.astype(jnp.int32)
    dval, hit = _sc_resolve(idx32, v)
    out = pl.pallas_call(
        _tc_body,
        grid=(_B, _NB),
        in_specs=[
            pl.BlockSpec((1, _B, _R), lambda b, r: (r, 0, 0)),
            pl.BlockSpec((1, _B, _R), lambda b, r: (r, 0, 0)),
            pl.BlockSpec((1, _R, _N), lambda b, r: (b, r, 0)),
        ],
        out_specs=pl.BlockSpec((1, _R, _N), lambda b, r: (b, r, 0)),
        out_shape=jax.ShapeDtypeStruct((_B, _N, _N), jnp.float32),
        compiler_params=pltpu.CompilerParams(
            dimension_semantics=("parallel", "parallel")),
    )(dval, hit, t)
    return out
